# baseline (speedup 1.0000x reference)
import functools

import jax
import jax.numpy as jnp
from jax import lax
from jax.experimental import pallas as pl
from jax.experimental.pallas import tpu as pltpu
from jax.experimental.pallas import tpu_sc as plsc

_B = 16384
_CTX = 20
_D = 64
_NE = 2 * _B
_NC = 2
_NS = 16
_NW = _NC * _NS
_HEPW = _B // _NW
_G = 32
_NGH = _HEPW // _G
_RPG = _G * _CTX
_CHUNK = 128
_NCHUNK = _RPG // _CHUNK

_MESH = plsc.VectorSubcoreMesh(core_axis_name="c", subcore_axis_name="s")
_PARAMS = pltpu.CompilerParams(use_tc_tiling_on_sc=False)


_GA = 16
_RPGA = _GA * _CTX
_EPW = 2 * _HEPW
_NGA = _EPW // _GA
_CHUNKS_A = (128, 128, 64)


def _pool_body(pu_hbm, nu_hbm, u_hbm, pooled_hbm,
               idxu_v, rows_a, rows_b, pooled_v, sem_a, sem_b):
    wid = lax.axis_index("s") * _NC + lax.axis_index("c")
    ebase = wid * _HEPW

    pltpu.sync_copy(pu_hbm.at[pl.ds(ebase * _CTX, _HEPW * _CTX)],
                    idxu_v.at[pl.ds(0, _HEPW * _CTX)])
    pltpu.sync_copy(nu_hbm.at[pl.ds(ebase * _CTX, _HEPW * _CTX)],
                    idxu_v.at[pl.ds(_HEPW * _CTX, _HEPW * _CTX)])

    def fire(g, rows_buf, sem):
        off = 0
        for n in _CHUNKS_A:
            pltpu.async_copy(
                u_hbm.at[idxu_v.at[pl.ds(g * _RPGA + off, n)]],
                rows_buf.at[pl.ds(off, n)],
                sem,
            )
            off += n

    def drain(g, rows_buf, sem):
        off = 0
        for n in _CHUNKS_A:
            pltpu.make_async_copy(
                u_hbm.at[idxu_v.at[pl.ds(g * _RPGA + off, n)]],
                rows_buf.at[pl.ds(off, n)],
                sem,
            ).wait()
            off += n

    def compute(g, rows_buf):
        @pl.loop(0, _GA)
        def elem_loop(e):
            row0 = e * _CTX
            accs = [jnp.zeros((16,), jnp.float32) for _ in range(_D // 16)]
            for r in range(_CTX):
                for c in range(_D // 16):
                    accs[c] = accs[c] + rows_buf[row0 + r, pl.ds(c * 16, 16)]
            for c in range(_D // 16):
                pooled_v[g * _GA + e, pl.ds(c * 16, 16)] = accs[c]

    fire(0, rows_a, sem_a)

    @pl.loop(0, _NGA // 2)
    def pair_loop(t):
        g0 = 2 * t
        g1 = g0 + 1
        fire(g1, rows_b, sem_b)
        drain(g0, rows_a, sem_a)
        compute(g0, rows_a)

        @pl.when(t < _NGA // 2 - 1)
        def _prefetch():
            fire(g0 + 2, rows_a, sem_a)

        drain(g1, rows_b, sem_b)
        compute(g1, rows_b)

    pltpu.sync_copy(pooled_v.at[pl.ds(0, _HEPW)],
                    pooled_hbm.at[pl.ds(ebase, _HEPW)])
    pltpu.sync_copy(pooled_v.at[pl.ds(_HEPW, _HEPW)],
                    pooled_hbm.at[pl.ds(_B + ebase, _HEPW)])


_pool = functools.partial(
    pl.kernel,
    out_type=jax.ShapeDtypeStruct((_NE, _D), jnp.float32),
    mesh=_MESH,
    scratch_types=[
        pltpu.VMEM((_EPW * _CTX,), jnp.int32),
        pltpu.VMEM((_RPGA, _D), jnp.float32),
        pltpu.VMEM((_RPGA, _D), jnp.float32),
        pltpu.VMEM((_EPW, _D), jnp.float32),
        pltpu.SemaphoreType.DMA,
        pltpu.SemaphoreType.DMA,
    ],
    compiler_params=_PARAMS,
)(_pool_body)


_GDN = lax.GatherDimensionNumbers(
    offset_dims=(), collapsed_slice_dims=(0,), start_index_map=(0,))


def _dot_body(pw_hbm, nw_hbm, w_hbm, pooled_hbm, scores_hbm,
              idxw_v, wrows_v, pooled_v, scores_v, sem, sem_p):
    wid = lax.axis_index("s") * _NC + lax.axis_index("c")
    ebase = wid * _HEPW
    lane = lax.iota(jnp.int32, 16)
    perms = [((lane ^ sh).astype(jnp.int32))[:, None] for sh in (8, 4, 2, 1)]

    for iw_hbm, obase in ((pw_hbm, 0), (nw_hbm, _B)):
        pltpu.sync_copy(iw_hbm.at[pl.ds(ebase, _HEPW)], idxw_v)
        chunk_cps = []
        for j in range(_HEPW // _CHUNK):
            sl = pl.ds(j * _CHUNK, _CHUNK)
            pcp = pltpu.async_copy(
                pooled_hbm.at[pl.ds(obase + ebase + j * _CHUNK, _CHUNK)],
                pooled_v.at[sl], sem_p)
            wcp = pltpu.async_copy(
                w_hbm.at[idxw_v.at[sl]], wrows_v.at[sl], sem)
            chunk_cps.append((pcp, wcp))

        for j, (pcp, wcp) in enumerate(chunk_cps):
            pcp.wait()
            wcp.wait()

            @pl.loop(0, _CHUNK // 16)
            def sub_loop(sg, base=j * _CHUNK):
                svec = jnp.zeros((16,), jnp.float32)
                for ei in range(16):
                    e = base + sg * 16 + ei
                    p = jnp.zeros((16,), jnp.float32)
                    for c in range(_D // 16):
                        p = p + (pooled_v[e, pl.ds(c * 16, 16)]
                                 * wrows_v[e, pl.ds(c * 16, 16)])
                    for perm in perms:
                        p = p + lax.gather(
                            p, perm, _GDN, (1,),
                            mode=lax.GatherScatterMode.PROMISE_IN_BOUNDS)
                    svec = jnp.where(lane == ei, p, svec)
                scores_v[pl.ds(base + sg * 16, 16)] = svec

        pltpu.sync_copy(scores_v, scores_hbm.at[pl.ds(obase + ebase, _HEPW)])


_dot = functools.partial(
    pl.kernel,
    out_type=jax.ShapeDtypeStruct((_NE,), jnp.float32),
    mesh=_MESH,
    scratch_types=[
        pltpu.VMEM((_HEPW,), jnp.int32),
        pltpu.VMEM((_HEPW, _D), jnp.float32),
        pltpu.VMEM((_HEPW, _D), jnp.float32),
        pltpu.VMEM((_HEPW,), jnp.float32),
        pltpu.SemaphoreType.DMA,
        pltpu.SemaphoreType.DMA,
    ],
    compiler_params=_PARAMS,
)(_dot_body)


def _tc_loss_body(scores_ref, out_ref):
    x = scores_ref[...]
    row = lax.broadcasted_iota(jnp.int32, x.shape, 0)
    y = jnp.where(row < _NE // 128 // 2, x, -x)
    ls = jnp.minimum(y, 0.0) - jnp.log1p(jnp.exp(-jnp.abs(y)))
    out_ref[0, 0] = -jnp.sum(ls)


_tc_loss = pl.pallas_call(
    _tc_loss_body,
    out_shape=jax.ShapeDtypeStruct((1, 1), jnp.float32),
    out_specs=pl.BlockSpec(memory_space=pltpu.SMEM),
)


@jax.jit
def kernel(pos_u, pos_w, neg_u, neg_w, u_weight, w_weight):
    pooled = _pool(pos_u.reshape(-1), neg_u.reshape(-1), u_weight)
    scores = _dot(pos_w, neg_w, w_weight, pooled)
    loss = _tc_loss(scores.reshape(_NE // 128, 128))
    return loss[0, 0]


# speedup vs baseline: 1.0250x; 1.0250x over previous
import functools

import jax
import jax.numpy as jnp
from jax import lax
from jax.experimental import pallas as pl
from jax.experimental.pallas import tpu as pltpu
from jax.experimental.pallas import tpu_sc as plsc

_B = 16384
_CTX = 20
_D = 64
_NE = 2 * _B
_NC = 2
_NS = 16
_NW = _NC * _NS
_HEPW = _B // _NW
_G = 32
_NGH = _HEPW // _G
_RPG = _G * _CTX
_CHUNK = 128
_NCHUNK = _RPG // _CHUNK

_MESH = plsc.VectorSubcoreMesh(core_axis_name="c", subcore_axis_name="s")
_PARAMS = pltpu.CompilerParams(use_tc_tiling_on_sc=False)


_GA = 16
_RPGA = _GA * _CTX
_EPW = 2 * _HEPW
_NGA = _EPW // _GA
_CHUNKS_A = (128, 128, 64)


def _pool_body(pu_hbm, nu_hbm, u_hbm, pooled_hbm,
               idxu_v, rows_a, rows_b, pooled_v, sem_a, sem_b):
    wid = lax.axis_index("s") * _NC + lax.axis_index("c")
    ebase = wid * _HEPW

    pltpu.sync_copy(pu_hbm.at[pl.ds(ebase * _CTX, _HEPW * _CTX)],
                    idxu_v.at[pl.ds(0, _HEPW * _CTX)])
    pltpu.sync_copy(nu_hbm.at[pl.ds(ebase * _CTX, _HEPW * _CTX)],
                    idxu_v.at[pl.ds(_HEPW * _CTX, _HEPW * _CTX)])

    def fire(g, rows_buf, sem):
        off = 0
        for n in _CHUNKS_A:
            pltpu.async_copy(
                u_hbm.at[idxu_v.at[pl.ds(g * _RPGA + off, n)]],
                rows_buf.at[pl.ds(off, n)],
                sem,
            )
            off += n

    def drain(g, rows_buf, sem):
        off = 0
        for n in _CHUNKS_A:
            pltpu.make_async_copy(
                u_hbm.at[idxu_v.at[pl.ds(g * _RPGA + off, n)]],
                rows_buf.at[pl.ds(off, n)],
                sem,
            ).wait()
            off += n

    def compute(g, rows_buf):
        @pl.loop(0, _GA)
        def elem_loop(e):
            row0 = e * _CTX
            accs = [jnp.zeros((16,), jnp.float32) for _ in range(_D // 16)]
            for r in range(_CTX):
                for c in range(_D // 16):
                    accs[c] = accs[c] + rows_buf[row0 + r, pl.ds(c * 16, 16)]
            for c in range(_D // 16):
                pooled_v[g * _GA + e, pl.ds(c * 16, 16)] = accs[c]

    fire(0, rows_a, sem_a)

    @pl.loop(0, _NGA // 2)
    def pair_loop(t):
        g0 = 2 * t
        g1 = g0 + 1
        fire(g1, rows_b, sem_b)
        drain(g0, rows_a, sem_a)
        compute(g0, rows_a)

        @pl.when(t < _NGA // 2 - 1)
        def _prefetch():
            fire(g0 + 2, rows_a, sem_a)

        drain(g1, rows_b, sem_b)
        compute(g1, rows_b)

    pltpu.sync_copy(pooled_v.at[pl.ds(0, _HEPW)],
                    pooled_hbm.at[pl.ds(ebase, _HEPW)])
    pltpu.sync_copy(pooled_v.at[pl.ds(_HEPW, _HEPW)],
                    pooled_hbm.at[pl.ds(_B + ebase, _HEPW)])


_pool = functools.partial(
    pl.kernel,
    out_type=jax.ShapeDtypeStruct((_NE, _D), jnp.float32),
    mesh=_MESH,
    scratch_types=[
        pltpu.VMEM((_EPW * _CTX,), jnp.int32),
        pltpu.VMEM((_RPGA, _D), jnp.float32),
        pltpu.VMEM((_RPGA, _D), jnp.float32),
        pltpu.VMEM((_EPW, _D), jnp.float32),
        pltpu.SemaphoreType.DMA,
        pltpu.SemaphoreType.DMA,
    ],
    compiler_params=_PARAMS,
)(_pool_body)


_GDN = lax.GatherDimensionNumbers(
    offset_dims=(), collapsed_slice_dims=(0,), start_index_map=(0,))


def _dot_body(pw_hbm, nw_hbm, w_hbm, pooled_hbm, scores_hbm,
              idxw_v, wrows_v, pooled_v, scores_v, sem, sem_p):
    wid = lax.axis_index("s") * _NC + lax.axis_index("c")
    ebase = wid * _HEPW
    lane = lax.iota(jnp.int32, 16)
    perms = [((lane ^ sh).astype(jnp.int32))[:, None] for sh in (8, 4, 2, 1)]

    for iw_hbm, obase in ((pw_hbm, 0), (nw_hbm, _B)):
        pltpu.sync_copy(iw_hbm.at[pl.ds(ebase, _HEPW)], idxw_v)
        pooled_cp = pltpu.async_copy(
            pooled_hbm.at[pl.ds(obase + ebase, _HEPW)], pooled_v, sem_p)
        copies = [
            pltpu.async_copy(
                w_hbm.at[idxw_v.at[pl.ds(j * _CHUNK, _CHUNK)]],
                wrows_v.at[pl.ds(j * _CHUNK, _CHUNK)],
                sem,
            )
            for j in range(_HEPW // _CHUNK)
        ]
        pooled_cp.wait()
        for c in copies:
            c.wait()

        @pl.loop(0, _HEPW // 16)
        def sub_loop(sg):
            svec = jnp.zeros((16,), jnp.float32)
            for ei in range(16):
                e = sg * 16 + ei
                p = jnp.zeros((16,), jnp.float32)
                for c in range(_D // 16):
                    p = p + (pooled_v[e, pl.ds(c * 16, 16)]
                             * wrows_v[e, pl.ds(c * 16, 16)])
                for perm in perms:
                    p = p + lax.gather(
                        p, perm, _GDN, (1,),
                        mode=lax.GatherScatterMode.PROMISE_IN_BOUNDS)
                svec = jnp.where(lane == ei, p, svec)
            scores_v[pl.ds(sg * 16, 16)] = svec

        pltpu.sync_copy(scores_v, scores_hbm.at[pl.ds(obase + ebase, _HEPW)])


_dot = functools.partial(
    pl.kernel,
    out_type=jax.ShapeDtypeStruct((_NE,), jnp.float32),
    mesh=_MESH,
    scratch_types=[
        pltpu.VMEM((_HEPW,), jnp.int32),
        pltpu.VMEM((_HEPW, _D), jnp.float32),
        pltpu.VMEM((_HEPW, _D), jnp.float32),
        pltpu.VMEM((_HEPW,), jnp.float32),
        pltpu.SemaphoreType.DMA,
        pltpu.SemaphoreType.DMA,
    ],
    compiler_params=_PARAMS,
)(_dot_body)


def _tc_loss_body(scores_ref, out_ref):
    x = scores_ref[...]
    row = lax.broadcasted_iota(jnp.int32, x.shape, 0)
    y = jnp.where(row < _NE // 128 // 2, x, -x)
    ls = jnp.minimum(y, 0.0) - jnp.log1p(jnp.exp(-jnp.abs(y)))
    out_ref[0, 0] = -jnp.sum(ls)


_tc_loss = pl.pallas_call(
    _tc_loss_body,
    out_shape=jax.ShapeDtypeStruct((1, 1), jnp.float32),
    out_specs=pl.BlockSpec(memory_space=pltpu.SMEM),
)


@jax.jit
def kernel(pos_u, pos_w, neg_u, neg_w, u_weight, w_weight):
    pooled = _pool(pos_u.reshape(-1), neg_u.reshape(-1), u_weight)
    scores = _dot(pos_w, neg_w, w_weight, pooled)
    loss = _tc_loss(scores.reshape(_NE // 128, 128))
    return loss[0, 0]


# speedup vs baseline: 1.0263x; 1.0012x over previous
import functools

import jax
import jax.numpy as jnp
from jax import lax
from jax.experimental import pallas as pl
from jax.experimental.pallas import tpu as pltpu
from jax.experimental.pallas import tpu_sc as plsc

_B = 16384
_CTX = 20
_D = 64
_NE = 2 * _B
_NC = 2
_NS = 16
_NW = _NC * _NS
_HEPW = _B // _NW
_CHUNK = 128

_MESH = plsc.VectorSubcoreMesh(core_axis_name="c", subcore_axis_name="s")
_PARAMS = pltpu.CompilerParams(use_tc_tiling_on_sc=False)


_GA = 16
_RPGA = _GA * _CTX
_EPW = 2 * _HEPW
_NGA = _EPW // _GA
_CHUNKS_A = (128, 128, 64)


def _pool_body(pu_hbm, nu_hbm, u_hbm, pooled_hbm,
               idxu_v, rows_a, rows_b, pooled_v, sem_a, sem_b):
    wid = lax.axis_index("s") * _NC + lax.axis_index("c")
    ebase = wid * _HEPW

    pltpu.sync_copy(pu_hbm.at[pl.ds(ebase * _CTX, _HEPW * _CTX)],
                    idxu_v.at[pl.ds(0, _HEPW * _CTX)])
    pltpu.sync_copy(nu_hbm.at[pl.ds(ebase * _CTX, _HEPW * _CTX)],
                    idxu_v.at[pl.ds(_HEPW * _CTX, _HEPW * _CTX)])

    def fire(g, rows_buf, sem):
        off = 0
        for n in _CHUNKS_A:
            pltpu.async_copy(
                u_hbm.at[idxu_v.at[pl.ds(g * _RPGA + off, n)]],
                rows_buf.at[pl.ds(off, n)],
                sem,
            )
            off += n

    def drain(g, rows_buf, sem):
        off = 0
        for n in _CHUNKS_A:
            pltpu.make_async_copy(
                u_hbm.at[idxu_v.at[pl.ds(g * _RPGA + off, n)]],
                rows_buf.at[pl.ds(off, n)],
                sem,
            ).wait()
            off += n

    def compute(g, rows_buf):
        @pl.loop(0, _GA)
        def elem_loop(e):
            row0 = e * _CTX
            accs = [jnp.zeros((16,), jnp.float32) for _ in range(_D // 16)]
            for r in range(_CTX):
                for c in range(_D // 16):
                    accs[c] = accs[c] + rows_buf[row0 + r, pl.ds(c * 16, 16)]
            for c in range(_D // 16):
                pooled_v[g * _GA + e, pl.ds(c * 16, 16)] = accs[c]

    fire(0, rows_a, sem_a)

    @pl.loop(0, _NGA // 2)
    def pair_loop(t):
        g0 = 2 * t
        g1 = g0 + 1
        fire(g1, rows_b, sem_b)
        drain(g0, rows_a, sem_a)
        compute(g0, rows_a)

        @pl.when(t < _NGA // 2 - 1)
        def _prefetch():
            fire(g0 + 2, rows_a, sem_a)

        drain(g1, rows_b, sem_b)
        compute(g1, rows_b)

    pltpu.sync_copy(pooled_v.at[pl.ds(0, _HEPW)],
                    pooled_hbm.at[pl.ds(ebase, _HEPW)])
    pltpu.sync_copy(pooled_v.at[pl.ds(_HEPW, _HEPW)],
                    pooled_hbm.at[pl.ds(_B + ebase, _HEPW)])


_pool = functools.partial(
    pl.kernel,
    out_type=jax.ShapeDtypeStruct((_NE, _D), jnp.float32),
    mesh=_MESH,
    scratch_types=[
        pltpu.VMEM((_EPW * _CTX,), jnp.int32),
        pltpu.VMEM((_RPGA, _D), jnp.float32),
        pltpu.VMEM((_RPGA, _D), jnp.float32),
        pltpu.VMEM((_EPW, _D), jnp.float32),
        pltpu.SemaphoreType.DMA,
        pltpu.SemaphoreType.DMA,
    ],
    compiler_params=_PARAMS,
)(_pool_body)


_GDN = lax.GatherDimensionNumbers(
    offset_dims=(), collapsed_slice_dims=(0,), start_index_map=(0,))


def _dot_body(pw_hbm, nw_hbm, w_hbm, pooled_hbm, scores_hbm,
              idxw_v, wrows_v, pooled_v, scores_v, sem, sem_p):
    wid = lax.axis_index("s") * _NC + lax.axis_index("c")
    ebase = wid * _HEPW
    lane = lax.iota(jnp.int32, 16)
    perms = [((lane ^ sh).astype(jnp.int32))[:, None] for sh in (8, 4, 2, 1)]

    for iw_hbm, obase in ((pw_hbm, 0), (nw_hbm, _B)):
        pltpu.sync_copy(iw_hbm.at[pl.ds(ebase, _HEPW)], idxw_v)
        pooled_cp = pltpu.async_copy(
            pooled_hbm.at[pl.ds(obase + ebase, _HEPW)], pooled_v, sem_p)
        copies = [
            pltpu.async_copy(
                w_hbm.at[idxw_v.at[pl.ds(j * _CHUNK, _CHUNK)]],
                wrows_v.at[pl.ds(j * _CHUNK, _CHUNK)],
                sem,
            )
            for j in range(_HEPW // _CHUNK)
        ]
        pooled_cp.wait()
        for c in copies:
            c.wait()

        @pl.loop(0, _HEPW // 16)
        def sub_loop(sg):
            svec = jnp.zeros((16,), jnp.float32)
            for ei in range(16):
                e = sg * 16 + ei
                p = jnp.zeros((16,), jnp.float32)
                for c in range(_D // 16):
                    p = p + (pooled_v[e, pl.ds(c * 16, 16)]
                             * wrows_v[e, pl.ds(c * 16, 16)])
                for perm in perms:
                    p = p + lax.gather(
                        p, perm, _GDN, (1,),
                        mode=lax.GatherScatterMode.PROMISE_IN_BOUNDS)
                svec = jnp.where(lane == ei, p, svec)
            scores_v[pl.ds(sg * 16, 16)] = svec

        pltpu.sync_copy(scores_v, scores_hbm.at[pl.ds(obase + ebase, _HEPW)])


_dot = functools.partial(
    pl.kernel,
    out_type=jax.ShapeDtypeStruct((_NE,), jnp.float32),
    mesh=_MESH,
    scratch_types=[
        pltpu.VMEM((_HEPW,), jnp.int32),
        pltpu.VMEM((_HEPW, _D), jnp.float32),
        pltpu.VMEM((_HEPW, _D), jnp.float32),
        pltpu.VMEM((_HEPW,), jnp.float32),
        pltpu.SemaphoreType.DMA,
        pltpu.SemaphoreType.DMA,
    ],
    compiler_params=_PARAMS,
)(_dot_body)


def _tc_loss_body(scores_ref, out_ref):
    x = scores_ref[...]
    row = lax.broadcasted_iota(jnp.int32, x.shape, 0)
    y = jnp.where(row < _NE // 128 // 2, x, -x)
    ls = jnp.minimum(y, 0.0) - jnp.log1p(jnp.exp(-jnp.abs(y)))
    out_ref[0, 0] = -jnp.sum(ls)


_tc_loss = pl.pallas_call(
    _tc_loss_body,
    out_shape=jax.ShapeDtypeStruct((1, 1), jnp.float32),
    out_specs=pl.BlockSpec(memory_space=pltpu.SMEM),
)


@jax.jit
def kernel(pos_u, pos_w, neg_u, neg_w, u_weight, w_weight):
    pooled = _pool(pos_u.reshape(-1), neg_u.reshape(-1), u_weight)
    scores = _dot(pos_w, neg_w, w_weight, pooled)
    loss = _tc_loss(scores.reshape(_NE // 128, 128))
    return loss[0, 0]
